# baseline (speedup 1.0000x reference)
import jax
import jax.numpy as jnp
from jax import lax
from jax.experimental import pallas as pl
from jax.experimental.pallas import tpu as pltpu
from jax.experimental.pallas import tpu_sc as plsc

N = 1024
C = 100000
L = 16
NS = 16
PER = N // NS


def _sc_body(probt_hbm, target_hbm, reward_hbm, out_hbm,
             tgt_v, rwd_v, win_v, part_v, acc_v, out_v, shared, sem):
    sid = lax.axis_index("s")
    base = sid * PER

    pltpu.sync_copy(target_hbm.at[pl.ds(base, PER)], tgt_v)
    pltpu.sync_copy(reward_hbm.at[pl.ds(base, PER)], rwd_v)

    c0 = pl.multiple_of((sid // 2) * 128, 128)
    off = (sid % 2) * PER
    pltpu.async_copy(
        probt_hbm.at[tgt_v, pl.ds(c0, 2 * PER)], win_v, sem).wait()

    lane = lax.iota(jnp.int32, L)

    def body(r, acc):
        j16 = (r // L) * L
        wv = win_v[r, pl.ds(off + j16, L)]
        rv = rwd_v[pl.ds(j16, L)]
        return acc + jnp.where(lane == r - j16, wv * rv, 0.0)

    part_v[...] = lax.fori_loop(0, PER, body, jnp.zeros((L,), jnp.float32))

    pltpu.sync_copy(part_v, shared.at[pl.ds(sid * L, L)])
    plsc.subcore_barrier()

    @pl.when(sid == 0)
    def _():
        pltpu.sync_copy(shared, acc_v)
        tot = lax.fori_loop(
            0, NS,
            lambda i, t: t + acc_v[pl.ds(i * L, L)],
            jnp.zeros((L,), jnp.float32))
        s = tot[0]
        for i in range(1, L):
            s = s + tot[i]
        out_v[...] = jnp.broadcast_to(-s, (L,))
        pltpu.sync_copy(out_v, out_hbm)


@jax.jit
def _sc_loss(probt, target, reward):
    mesh = plsc.VectorSubcoreMesh(
        core_axis_name="c", subcore_axis_name="s", num_cores=1, num_subcores=NS)
    run = pl.kernel(
        _sc_body,
        out_type=jax.ShapeDtypeStruct((L,), jnp.float32),
        mesh=mesh,
        scratch_types=[
            pltpu.VMEM((PER,), jnp.int32),
            pltpu.VMEM((PER,), jnp.float32),
            pltpu.VMEM((PER, 2 * PER), jnp.float32),
            pltpu.VMEM((L,), jnp.float32),
            pltpu.VMEM((NS * L,), jnp.float32),
            pltpu.VMEM((L,), jnp.float32),
            pltpu.VMEM_SHARED((NS * L,), jnp.float32),
            pltpu.SemaphoreType.DMA,
        ],
    )
    return run(probt, target, reward)


def kernel(prob, target, reward):
    out = _sc_loss(prob.T, target.astype(jnp.int32), reward)
    return out[0]


# speedup vs baseline: 1.0229x; 1.0229x over previous
import jax
import jax.numpy as jnp
from jax import lax
from jax.experimental import pallas as pl
from jax.experimental.pallas import tpu as pltpu
from jax.experimental.pallas import tpu_sc as plsc

N = 1024
C = 100000
L = 16
NS = 16
PER = N // NS


def _sc_body(probt_hbm, target_hbm, reward_hbm, out_hbm,
             tgt_v, rwd_v, win_v, part_v, acc_v, out_v, shared,
             sem_t, sem_r, sem_g):
    sid = lax.axis_index("s")
    base = sid * PER

    ct = pltpu.async_copy(target_hbm.at[pl.ds(base, PER)], tgt_v, sem_t)
    cr = pltpu.async_copy(reward_hbm.at[pl.ds(base, PER)], rwd_v, sem_r)
    ct.wait()

    c0 = pl.multiple_of((sid // 2) * 128, 128)
    off = (sid % 2) * PER
    cg = pltpu.async_copy(probt_hbm.at[tgt_v, pl.ds(c0, 2 * PER)], win_v, sem_g)
    cr.wait()
    cg.wait()

    lane = lax.iota(jnp.int32, L)

    def body(r, acc):
        j16 = (r // L) * L
        wv = win_v[r, pl.ds(off + j16, L)]
        rv = rwd_v[pl.ds(j16, L)]
        return acc + jnp.where(lane == r - j16, wv * rv, 0.0)

    part_v[...] = lax.fori_loop(0, PER, body, jnp.zeros((L,), jnp.float32))

    pltpu.sync_copy(part_v, shared.at[pl.ds(sid * L, L)])
    plsc.subcore_barrier()

    @pl.when(sid == 0)
    def _():
        pltpu.sync_copy(shared, acc_v)
        tot = lax.fori_loop(
            0, NS,
            lambda i, t: t + acc_v[pl.ds(i * L, L)],
            jnp.zeros((L,), jnp.float32))
        s = tot[0]
        for i in range(1, L):
            s = s + tot[i]
        out_v[...] = jnp.broadcast_to(-s, (L,))
        pltpu.sync_copy(out_v, out_hbm)


@jax.jit
def _sc_loss(probt, target, reward):
    mesh = plsc.VectorSubcoreMesh(
        core_axis_name="c", subcore_axis_name="s", num_cores=1, num_subcores=NS)
    run = pl.kernel(
        _sc_body,
        out_type=jax.ShapeDtypeStruct((L,), jnp.float32),
        mesh=mesh,
        scratch_types=[
            pltpu.VMEM((PER,), jnp.int32),
            pltpu.VMEM((PER,), jnp.float32),
            pltpu.VMEM((PER, 2 * PER), jnp.float32),
            pltpu.VMEM((L,), jnp.float32),
            pltpu.VMEM((NS * L,), jnp.float32),
            pltpu.VMEM((L,), jnp.float32),
            pltpu.VMEM_SHARED((NS * L,), jnp.float32),
            pltpu.SemaphoreType.DMA,
            pltpu.SemaphoreType.DMA,
            pltpu.SemaphoreType.DMA,
        ],
    )
    return run(probt, target, reward)


def kernel(prob, target, reward):
    out = _sc_loss(prob.T, target.astype(jnp.int32), reward)
    return out[0]


# speedup vs baseline: 1.0290x; 1.0059x over previous
import jax
import jax.numpy as jnp
from jax import lax
from jax.experimental import pallas as pl
from jax.experimental.pallas import tpu as pltpu
from jax.experimental.pallas import tpu_sc as plsc

N = 1024
C = 100000
L = 16
NS = 16
PER = N // NS


def _sc_body(probt_hbm, target_hbm, reward_hbm, out_hbm,
             tgt_v, rwd_v, win_v, part_v, acc_v, out_v, shared,
             sem_t, sem_r, sem_g):
    sid = lax.axis_index("s")
    base = sid * PER

    ct = pltpu.async_copy(target_hbm.at[pl.ds(base, PER)], tgt_v, sem_t)
    cr = pltpu.async_copy(reward_hbm.at[pl.ds(base, PER)], rwd_v, sem_r)
    ct.wait()

    c0 = pl.multiple_of((sid // 2) * 128, 128)
    off = (sid % 2) * PER
    cg = pltpu.async_copy(probt_hbm.at[tgt_v, pl.ds(c0, 2 * PER)], win_v, sem_g)
    cr.wait()
    cg.wait()

    lane = lax.iota(jnp.int32, L)
    zero = jnp.zeros((L,), jnp.float32)

    def body(r0, accs):
        j16 = (r0 // L) * L
        rv = rwd_v[pl.ds(j16, L)]
        out = []
        for u in range(4):
            r = r0 + u
            wv = win_v[r, pl.ds(off + j16, L)]
            out.append(accs[u] + jnp.where(lane == r - j16, wv * rv, 0.0))
        return tuple(out)

    a0, a1, a2, a3 = plsc.parallel_loop(
        0, PER, 4, carry=(zero, zero, zero, zero))(body)
    part_v[...] = (a0 + a1) + (a2 + a3)

    pltpu.sync_copy(part_v, shared.at[pl.ds(sid * L, L)])
    plsc.subcore_barrier()

    @pl.when(sid == 0)
    def _():
        pltpu.sync_copy(shared, acc_v)
        tot = lax.fori_loop(
            0, NS,
            lambda i, t: t + acc_v[pl.ds(i * L, L)],
            jnp.zeros((L,), jnp.float32))
        s = tot[0]
        for i in range(1, L):
            s = s + tot[i]
        out_v[...] = jnp.broadcast_to(-s, (L,))
        pltpu.sync_copy(out_v, out_hbm)


@jax.jit
def _sc_loss(probt, target, reward):
    mesh = plsc.VectorSubcoreMesh(
        core_axis_name="c", subcore_axis_name="s", num_cores=1, num_subcores=NS)
    run = pl.kernel(
        _sc_body,
        out_type=jax.ShapeDtypeStruct((L,), jnp.float32),
        mesh=mesh,
        scratch_types=[
            pltpu.VMEM((PER,), jnp.int32),
            pltpu.VMEM((PER,), jnp.float32),
            pltpu.VMEM((PER, 2 * PER), jnp.float32),
            pltpu.VMEM((L,), jnp.float32),
            pltpu.VMEM((NS * L,), jnp.float32),
            pltpu.VMEM((L,), jnp.float32),
            pltpu.VMEM_SHARED((NS * L,), jnp.float32),
            pltpu.SemaphoreType.DMA,
            pltpu.SemaphoreType.DMA,
            pltpu.SemaphoreType.DMA,
        ],
    )
    return run(probt, target, reward)


def kernel(prob, target, reward):
    out = _sc_loss(prob.T, target.astype(jnp.int32), reward)
    return out[0]


# speedup vs baseline: 1.6203x; 1.5746x over previous
import jax
import jax.numpy as jnp
from jax import lax
from jax.experimental import pallas as pl
from jax.experimental.pallas import tpu as pltpu

N = 1024
C = 100000
WB = 128


def _tc_body(target_sm, probt_any, reward_vm, out_vm, win_v, sem):
    def issue(i, _):
        t = target_sm[i]
        c0 = pl.multiple_of((i // WB) * WB, WB)
        pltpu.make_async_copy(
            probt_any.at[pl.ds(t, 1), pl.ds(c0, WB)],
            win_v.at[pl.ds(i, 1), :], sem).start()
        return 0

    lax.fori_loop(0, N, issue, 0, unroll=8)
    pltpu.make_async_copy(
        probt_any.at[pl.ds(0, N), pl.ds(0, WB)], win_v, sem).wait()

    rows = lax.broadcasted_iota(jnp.int32, (N, WB), 0)
    lanes = lax.broadcasted_iota(jnp.int32, (N, WB), 1)
    sel = jnp.where(lanes == rows % WB, win_v[...], 0.0)
    loss = jnp.sum(jnp.sum(sel, axis=1) * reward_vm[...])
    out_vm[...] = jnp.full((1, 1), -loss, jnp.float32)


@jax.jit
def _tc_loss(probt, target, reward):
    return pl.pallas_call(
        _tc_body,
        grid_spec=pltpu.PrefetchScalarGridSpec(
            num_scalar_prefetch=1,
            grid=(),
            in_specs=[
                pl.BlockSpec(memory_space=pl.ANY),
                pl.BlockSpec(memory_space=pltpu.VMEM),
            ],
            out_specs=pl.BlockSpec(memory_space=pltpu.VMEM),
            scratch_shapes=[
                pltpu.VMEM((N, WB), jnp.float32),
                pltpu.SemaphoreType.DMA,
            ],
        ),
        out_shape=jax.ShapeDtypeStruct((1, 1), jnp.float32),
    )(target, probt, reward)


def kernel(prob, target, reward):
    out = _tc_loss(prob.T, target.astype(jnp.int32), reward)
    return out[0, 0]


# speedup vs baseline: 1.6533x; 1.0204x over previous
import jax
import jax.numpy as jnp
from jax import lax
from jax.experimental import pallas as pl
from jax.experimental.pallas import tpu as pltpu

N = 1024
C = 100000
WB = 128


def _tc_body(target_sm, probt_any, reward_vm, out_vm, win_v, sem):
    def issue(i, _):
        t = target_sm[i]
        c0 = pl.multiple_of((i // WB) * WB, WB)
        pltpu.make_async_copy(
            probt_any.at[pl.ds(t, 1), pl.ds(c0, WB)],
            win_v.at[pl.ds(i, 1), :], sem).start()
        return 0

    lax.fori_loop(0, N, issue, 0, unroll=16)
    pltpu.make_async_copy(
        probt_any.at[pl.ds(0, N), pl.ds(0, WB)], win_v, sem).wait()

    rows = lax.broadcasted_iota(jnp.int32, (N, WB), 0)
    lanes = lax.broadcasted_iota(jnp.int32, (N, WB), 1)
    sel = jnp.where(lanes == rows % WB, win_v[...], 0.0)
    loss = jnp.sum(jnp.sum(sel, axis=1) * reward_vm[...])
    out_vm[...] = jnp.full((1, 1), -loss, jnp.float32)


@jax.jit
def _tc_loss(probt, target, reward):
    return pl.pallas_call(
        _tc_body,
        grid_spec=pltpu.PrefetchScalarGridSpec(
            num_scalar_prefetch=1,
            grid=(),
            in_specs=[
                pl.BlockSpec(memory_space=pl.ANY),
                pl.BlockSpec(memory_space=pltpu.VMEM),
            ],
            out_specs=pl.BlockSpec(memory_space=pltpu.VMEM),
            scratch_shapes=[
                pltpu.VMEM((N, WB), jnp.float32),
                pltpu.SemaphoreType.DMA,
            ],
        ),
        out_shape=jax.ShapeDtypeStruct((1, 1), jnp.float32),
    )(target, probt, reward)


def kernel(prob, target, reward):
    out = _tc_loss(prob.T, target.astype(jnp.int32), reward)
    return out[0, 0]


# speedup vs baseline: 2.3216x; 1.4042x over previous
import jax
import jax.numpy as jnp
from jax import lax
from jax.experimental import pallas as pl
from jax.experimental.pallas import tpu as pltpu

N = 1024
C = 100000
WB = 128
NCH = 8
CH = N // NCH


def _tc_body(target_sm, probt_any, reward_vm, out_vm, win_v, sems):
    for c in range(NCH):
        c0 = c * WB

        def issue(k, _, c=c, c0=c0):
            i = c * CH + k
            t = target_sm[i]
            pltpu.make_async_copy(
                probt_any.at[pl.ds(t, 1), pl.ds(c0, WB)],
                win_v.at[pl.ds(i, 1), :], sems.at[c]).start()
            return 0

        lax.fori_loop(0, CH, issue, 0, unroll=16)

    lanes = lax.broadcasted_iota(jnp.int32, (CH, WB), 1)
    rows = lax.broadcasted_iota(jnp.int32, (CH, WB), 0)
    mask = lanes == rows
    loss = jnp.float32(0.0)
    for c in range(NCH):
        pltpu.make_async_copy(
            probt_any.at[pl.ds(0, CH), pl.ds(0, WB)],
            win_v.at[pl.ds(c * CH, CH), :], sems.at[c]).wait()
        w = win_v[pl.ds(c * CH, CH), :]
        r = reward_vm[pl.ds(c * CH, CH)]
        loss = loss + jnp.sum(jnp.where(mask, w, 0.0).sum(axis=1) * r)
    out_vm[...] = jnp.full((1, 1), -loss, jnp.float32)


@jax.jit
def _tc_loss(probt, target, reward):
    return pl.pallas_call(
        _tc_body,
        grid_spec=pltpu.PrefetchScalarGridSpec(
            num_scalar_prefetch=1,
            grid=(),
            in_specs=[
                pl.BlockSpec(memory_space=pl.ANY),
                pl.BlockSpec(memory_space=pltpu.VMEM),
            ],
            out_specs=pl.BlockSpec(memory_space=pltpu.VMEM),
            scratch_shapes=[
                pltpu.VMEM((N, WB), jnp.float32),
                pltpu.SemaphoreType.DMA((NCH,)),
            ],
        ),
        out_shape=jax.ShapeDtypeStruct((1, 1), jnp.float32),
    )(target, probt, reward)


def kernel(prob, target, reward):
    out = _tc_loss(prob.T, target.astype(jnp.int32), reward)
    return out[0, 0]


# speedup vs baseline: 2.3734x; 1.0223x over previous
import jax
import jax.numpy as jnp
from jax import lax
from jax.experimental import pallas as pl
from jax.experimental.pallas import tpu as pltpu

N = 1024
C = 100000
WB = 128
NCH = 8
CH = N // NCH


def _tc_body(target_sm, probt_any, reward_vm, out_vm, win_v, sems):
    for c in range(NCH):
        c0 = c * WB

        def issue(k, _, c=c, c0=c0):
            i = c * CH + k
            t = target_sm[i]
            pltpu.make_async_copy(
                probt_any.at[pl.ds(t, 1), pl.ds(c0, WB)],
                win_v.at[pl.ds(i, 1), :], sems.at[c]).start()
            return 0

        lax.fori_loop(0, CH, issue, 0, unroll=32)

    lanes = lax.broadcasted_iota(jnp.int32, (CH, WB), 1)
    rows = lax.broadcasted_iota(jnp.int32, (CH, WB), 0)
    mask = lanes == rows
    loss = jnp.float32(0.0)
    for c in range(NCH):
        pltpu.make_async_copy(
            probt_any.at[pl.ds(0, CH), pl.ds(0, WB)],
            win_v.at[pl.ds(c * CH, CH), :], sems.at[c]).wait()
        w = win_v[pl.ds(c * CH, CH), :]
        r = reward_vm[pl.ds(c * CH, CH)]
        loss = loss + jnp.sum(jnp.where(mask, w, 0.0).sum(axis=1) * r)
    out_vm[...] = jnp.full((1, 1), -loss, jnp.float32)


@jax.jit
def _tc_loss(probt, target, reward):
    return pl.pallas_call(
        _tc_body,
        grid_spec=pltpu.PrefetchScalarGridSpec(
            num_scalar_prefetch=1,
            grid=(),
            in_specs=[
                pl.BlockSpec(memory_space=pl.ANY),
                pl.BlockSpec(memory_space=pltpu.VMEM),
            ],
            out_specs=pl.BlockSpec(memory_space=pltpu.VMEM),
            scratch_shapes=[
                pltpu.VMEM((N, WB), jnp.float32),
                pltpu.SemaphoreType.DMA((NCH,)),
            ],
        ),
        out_shape=jax.ShapeDtypeStruct((1, 1), jnp.float32),
    )(target, probt, reward)


def kernel(prob, target, reward):
    out = _tc_loss(prob.T, target.astype(jnp.int32), reward)
    return out[0, 0]


# speedup vs baseline: 2.4873x; 1.0480x over previous
import jax
import jax.numpy as jnp
from jax import lax
from jax.experimental import pallas as pl
from jax.experimental.pallas import tpu as pltpu

N = 1024
C = 100000
WB = 128
NCH = 8
CH = N // NCH


def _tc_body(target_sm, probt_any, reward_vm, out_vm, win_v, sems):
    for c in range(NCH):
        c0 = c * WB
        for k in range(CH):
            i = c * CH + k
            t = target_sm[i]
            pltpu.make_async_copy(
                probt_any.at[pl.ds(t, 1), pl.ds(c0, WB)],
                win_v.at[pl.ds(i, 1), :], sems.at[c]).start()

    lanes = lax.broadcasted_iota(jnp.int32, (CH, WB), 1)
    rows = lax.broadcasted_iota(jnp.int32, (CH, WB), 0)
    mask = lanes == rows
    loss = jnp.float32(0.0)
    for c in range(NCH):
        pltpu.make_async_copy(
            probt_any.at[pl.ds(0, CH), pl.ds(0, WB)],
            win_v.at[pl.ds(c * CH, CH), :], sems.at[c]).wait()
        w = win_v[pl.ds(c * CH, CH), :]
        r = reward_vm[pl.ds(c * CH, CH)]
        loss = loss + jnp.sum(jnp.where(mask, w, 0.0).sum(axis=1) * r)
    out_vm[...] = jnp.full((1, 1), -loss, jnp.float32)


@jax.jit
def _tc_loss(probt, target, reward):
    return pl.pallas_call(
        _tc_body,
        grid_spec=pltpu.PrefetchScalarGridSpec(
            num_scalar_prefetch=1,
            grid=(),
            in_specs=[
                pl.BlockSpec(memory_space=pl.ANY),
                pl.BlockSpec(memory_space=pltpu.VMEM),
            ],
            out_specs=pl.BlockSpec(memory_space=pltpu.VMEM),
            scratch_shapes=[
                pltpu.VMEM((N, WB), jnp.float32),
                pltpu.SemaphoreType.DMA((NCH,)),
            ],
        ),
        out_shape=jax.ShapeDtypeStruct((1, 1), jnp.float32),
    )(target, probt, reward)


def kernel(prob, target, reward):
    out = _tc_loss(prob.T, target.astype(jnp.int32), reward)
    return out[0, 0]
